# Initial kernel scaffold; baseline (speedup 1.0000x reference)
import jax
import jax.numpy as jnp
from jax.experimental import pallas as pl


def kernel(x, edge_index, edge_attr, batch, params):
    raise NotImplementedError("write your pallas kernel here")


# speedup vs baseline: 1.0024x; 1.0024x over previous
import jax
import jax.numpy as jnp
from jax.experimental import pallas as pl


def _linear(x, p):
    return x @ p["w"].T + p["b"]


def _bnorm(x, p):
    return p["g"] * (x - p["m"]) / jnp.sqrt(p["v"] + 1e-5) + p["b"]


def _head_body(g_ref, w1, b1, w2, b2, wp1, bp1, wp2, bp2, wc, bc,
               emb_ref, proj_ref, logits_ref):
    g = g_ref[...]
    emb = jnp.maximum(g @ w1[...].T + b1[...], 0.0)
    emb = emb @ w2[...].T + b2[...]
    emb_ref[...] = emb
    p = jnp.maximum(emb @ wp1[...].T + bp1[...], 0.0)
    proj_ref[...] = p @ wp2[...].T + bp2[...]
    logits_ref[...] = emb @ wc[...].T + bc[...]


def kernel(x, edge_index, edge_attr, batch, params):
    src = edge_index[0]
    dst = edge_index[1]
    N = x.shape[0]
    G = 16
    h = x
    for conv, bn in zip(params["convs"], params["bns"]):
        e = _linear(edge_attr, conv["edge_lin"])
        msg = jax.nn.relu(h[src] + e)
        agg = jax.ops.segment_sum(msg, dst, num_segments=N)
        z = h + agg
        z = _linear(z, conv["mlp_lin1"])
        z = _bnorm(z, conv["mlp_bn"])
        z = jax.nn.relu(z)
        z = _linear(z, conv["mlp_lin2"])
        h = jax.nn.relu(_bnorm(z, bn))
    counts = jax.ops.segment_sum(jnp.ones((N,), jnp.float32), batch, num_segments=G)
    sum_pool = jax.ops.segment_sum(h, batch, num_segments=G)
    mean_pool = sum_pool / jnp.maximum(counts, 1.0)[:, None]
    max_pool = jax.ops.segment_max(h, batch, num_segments=G)
    g = jnp.concatenate([mean_pool, max_pool, sum_pool], axis=1)

    p = params
    emb, proj, logits = pl.pallas_call(
        _head_body,
        out_shape=(
            jax.ShapeDtypeStruct((G, 128), jnp.float32),
            jax.ShapeDtypeStruct((G, 64), jnp.float32),
            jax.ShapeDtypeStruct((G, 10), jnp.float32),
        ),
    )(g, p["emb1"]["w"], p["emb1"]["b"], p["emb2"]["w"], p["emb2"]["b"],
      p["proj1"]["w"], p["proj1"]["b"], p["proj2"]["w"], p["proj2"]["b"],
      p["cls"]["w"], p["cls"]["b"])
    return (emb, proj, logits)


# speedup vs baseline: 2.9835x; 2.9763x over previous
import functools

import jax
import jax.numpy as jnp
from jax import lax
from jax.experimental import pallas as pl
from jax.experimental.pallas import tpu as pltpu
from jax.experimental.pallas import tpu_sc as plsc

N = 10000
E = 320000
DE = 16
H = 64
G = 16
NCORES = 2
NSUB = 16
NW = NCORES * NSUB
EPW = E // NW
B = 80
NJ = EPW // B
JB = 25
NJB = NJ // JB
RPT = N // NW
RSUB = N // NSUB


def _make_mp(C):
    CK = C // 16
    inplace = C == 128
    mesh = plsc.VectorSubcoreMesh(core_axis_name="c", subcore_axis_name="s")

    scratch = [
        pltpu.VMEM((JB, B), jnp.int32),
        pltpu.VMEM((JB, B), jnp.int32),
        pltpu.VMEM((B, C), jnp.float32),
        pltpu.VMEM((B, 128), jnp.float32),
        pltpu.VMEM_SHARED((N, 128), jnp.float32),
        pltpu.SemaphoreType.DMA,
    ]
    if not inplace:
        scratch.insert(4, pltpu.VMEM((B, 128), jnp.float32))

    @functools.partial(
        pl.kernel,
        out_type=jax.ShapeDtypeStruct((NCORES, N, 128), jnp.float32),
        mesh=mesh,
        scratch_types=scratch,
    )
    def mp(h_hbm, e_hbm, src_hbm, dst_hbm, out_hbm,
           src_v, dst_v, ebuf, hbuf, *rest):
        if inplace:
            agg_sh, sem = rest
            mbuf = hbuf
        else:
            mbuf, agg_sh, sem = rest
        c = lax.axis_index("c")
        s = lax.axis_index("s")
        wid = s * NCORES + c

        zv = jnp.zeros((16,), jnp.float32)

        def zrow(r, carry):
            for k in range(8):
                mbuf[r, pl.ds(k * 16, 16)] = zv
            return carry

        lax.fori_loop(0, B, zrow, 0)
        for t in range(8):
            b = s + NSUB * t

            @pl.when(b < N // B)
            def _():
                pltpu.sync_copy(mbuf, agg_sh.at[pl.ds(b * B, B)])
        plsc.subcore_barrier()

        ebase = wid * EPW

        def superstep(sb, carry):
            pltpu.sync_copy(src_hbm.at[wid, sb], src_v)
            pltpu.sync_copy(dst_hbm.at[wid, sb], dst_v)

            def step(j, c1):
                eoff = ebase + sb * (JB * B) + j * B
                pltpu.sync_copy(e_hbm.at[pl.ds(eoff, B)], ebuf)
                pltpu.async_copy(h_hbm.at[src_v.at[j]], hbuf, sem).wait()

                def crow(r, c2):
                    for k in range(CK):
                        sl = pl.ds(k * 16, 16)
                        mbuf[r, sl] = jnp.maximum(hbuf[r, sl] + ebuf[r, sl],
                                                  0.0)
                    return c2

                lax.fori_loop(0, B, crow, 0)
                pltpu.sync_copy(mbuf, agg_sh.at[dst_v.at[j]], add=True)
                return c1

            lax.fori_loop(0, JB, step, 0)
            return carry

        lax.fori_loop(0, NJB, superstep, 0)
        plsc.subcore_barrier()
        for t in range(8):
            b = s + NSUB * t

            @pl.when(b < N // B)
            def _():
                sl = pl.ds(b * B, B)
                pltpu.sync_copy(agg_sh.at[sl], out_hbm.at[c].at[sl])

    return mp


_mp128 = _make_mp(128)
_mp64 = _make_mp(64)


def _dot(a, b_mat):
    return lax.dot_general(a, b_mat, (((1,), (1,)), ((), ())),
                           preferred_element_type=jnp.float32)


def _elin_body(ea_ref, w0, b0, w1, b1, w2, b2, e0_ref, e1_ref, e2_ref):
    ea = ea_ref[...]
    e0_ref[...] = _dot(ea, w0[...]) + b0[...]
    e1_ref[...] = _dot(ea, w1[...]) + b1[...]
    e2_ref[...] = _dot(ea, w2[...]) + b2[...]


def _edge_linears(edge_attr, ws, bs):
    EC = 3200
    grid = E // EC
    full = lambda shp: pl.BlockSpec(shp, lambda i: (0,) * len(shp))
    return pl.pallas_call(
        _elin_body,
        grid=(grid,),
        in_specs=[
            pl.BlockSpec((EC, DE), lambda i: (i, 0)),
            full(ws[0].shape), full(bs[0].shape),
            full(ws[1].shape), full(bs[1].shape),
            full(ws[2].shape), full(bs[2].shape),
        ],
        out_specs=[
            pl.BlockSpec((EC, 128), lambda i: (i, 0)),
            pl.BlockSpec((EC, 64), lambda i: (i, 0)),
            pl.BlockSpec((EC, 64), lambda i: (i, 0)),
        ],
        out_shape=[
            jax.ShapeDtypeStruct((E, 128), jnp.float32),
            jax.ShapeDtypeStruct((E, 64), jnp.float32),
            jax.ShapeDtypeStruct((E, 64), jnp.float32),
        ],
    )(edge_attr, ws[0], bs[0], ws[1], bs[1], ws[2], bs[2])


def _make_mlp_body(C, pad):
    def _mlp_body(h_ref, agg_ref, w1, b1, w2, b2, out_ref):
        z = h_ref[:, :C] + agg_ref[0, :, :C] + agg_ref[1, :, :C]
        z = jnp.maximum(_dot(z, w1[...]) + b1[...], 0.0)
        o = jnp.maximum(_dot(z, w2[...]) + b2[...], 0.0)
        if pad:
            out_ref[...] = jnp.concatenate(
                [o, jnp.zeros((o.shape[0], 128 - H), jnp.float32)], axis=1)
        else:
            out_ref[...] = o
    return _mlp_body


def _node_mlp(h, agg, w1, b1, w2, b2, pad, C):
    HW = h.shape[1]
    OW = 128 if pad else H
    NC_ = 2000
    grid = N // NC_
    full = lambda shp: pl.BlockSpec(shp, lambda i: (0,) * len(shp))
    return pl.pallas_call(
        _make_mlp_body(C, pad),
        grid=(grid,),
        in_specs=[
            pl.BlockSpec((NC_, HW), lambda i: (i, 0)),
            pl.BlockSpec((NCORES, NC_, 128), lambda i: (0, i, 0)),
            full(w1.shape), full(b1.shape),
            full(w2.shape), full(b2.shape),
        ],
        out_specs=pl.BlockSpec((NC_, OW), lambda i: (i, 0)),
        out_shape=jax.ShapeDtypeStruct((N, OW), jnp.float32),
    )(h, agg, w1, b1, w2, b2)


def _pool_head_body(h_ref, batch_ref, w1, b1, w2, b2, wp1, bp1, wp2, bp2,
                    wc, bc, emb_ref, proj_ref, logits_ref):
    h = h_ref[...]
    bat = batch_ref[...]
    onehot = (bat == lax.broadcasted_iota(jnp.int32, (N, G), 1)
              ).astype(jnp.float32)
    sum_pool = lax.dot_general(onehot, h, (((0,), (0,)), ((), ())),
                               preferred_element_type=jnp.float32)
    counts = jnp.sum(onehot, axis=0)
    mean_pool = sum_pool / jnp.maximum(counts, 1.0)[:, None]

    neg = jnp.float32(-jnp.inf)
    CH = 500
    mx = jnp.full((G, H), neg, jnp.float32)
    for t in range(N // CH):
        hc = h[t * CH:(t + 1) * CH]
        oc = onehot[t * CH:(t + 1) * CH]
        masked = jnp.where(oc[:, :, None] > 0.0, hc[:, None, :], neg)
        mx = jnp.maximum(mx, jnp.max(masked, axis=0))
    g = jnp.concatenate([mean_pool, mx, sum_pool], axis=1)

    emb = jnp.maximum(_dot(g, w1[...]) + b1[...], 0.0)
    emb = _dot(emb, w2[...]) + b2[...]
    emb_ref[...] = emb
    p = jnp.maximum(_dot(emb, wp1[...]) + bp1[...], 0.0)
    proj_ref[...] = _dot(p, wp2[...]) + bp2[...]
    logits_ref[...] = _dot(emb, wc[...]) + bc[...]


def _pool_head(h, batch, p):
    args = (h, batch.reshape(N, 1),
            p["emb1"]["w"], p["emb1"]["b"].reshape(1, -1),
            p["emb2"]["w"], p["emb2"]["b"].reshape(1, -1),
            p["proj1"]["w"], p["proj1"]["b"].reshape(1, -1),
            p["proj2"]["w"], p["proj2"]["b"].reshape(1, -1),
            p["cls"]["w"], p["cls"]["b"].reshape(1, -1))
    return pl.pallas_call(
        _pool_head_body,
        out_shape=(
            jax.ShapeDtypeStruct((G, 128), jnp.float32),
            jax.ShapeDtypeStruct((G, 64), jnp.float32),
            jax.ShapeDtypeStruct((G, 10), jnp.float32),
        ),
    )(*args)


def _fold_bn(w, b, bn):
    scale = bn["g"] / jnp.sqrt(bn["v"] + 1e-5)
    wf = w * scale[:, None]
    bf = scale * (b - bn["m"]) + bn["b"]
    return wf, bf.reshape(1, -1)


def kernel(x, edge_index, edge_attr, batch, params):
    src = edge_index[0].reshape(NW, NJB, JB, B)
    dst = edge_index[1].reshape(NW, NJB, JB, B)

    convs = params["convs"]
    bns = params["bns"]
    e0, e1, e2 = _edge_linears(
        edge_attr,
        [convs[i]["edge_lin"]["w"] for i in range(3)],
        [convs[i]["edge_lin"]["b"].reshape(1, -1) for i in range(3)],
    )
    es = [e0, e1, e2]
    mps = [_mp128, _mp64, _mp64]

    h = x
    for i in range(3):
        w1, b1 = _fold_bn(convs[i]["mlp_lin1"]["w"], convs[i]["mlp_lin1"]["b"],
                          convs[i]["mlp_bn"])
        w2, b2 = _fold_bn(convs[i]["mlp_lin2"]["w"], convs[i]["mlp_lin2"]["b"],
                          bns[i])
        agg = mps[i](h, es[i], src, dst)
        h = _node_mlp(h, agg, w1, b1, w2, b2, pad=(i < 2),
                      C=(128 if i == 0 else 64))

    return _pool_head(h, batch, params)


# speedup vs baseline: 3.9547x; 1.3255x over previous
import functools

import jax
import jax.numpy as jnp
from jax import lax
from jax.experimental import pallas as pl
from jax.experimental.pallas import tpu as pltpu
from jax.experimental.pallas import tpu_sc as plsc

N = 10000
E = 320000
DE = 16
H = 64
G = 16
NCORES = 2
NSUB = 16
NW = NCORES * NSUB
EPW = E // NW
B = 80
NJ = EPW // B
JB = 25
NJB = NJ // JB
RPT = N // NW
RSUB = N // NSUB


def _make_mp(C):
    CK = C // 16
    mesh = plsc.VectorSubcoreMesh(core_axis_name="c", subcore_axis_name="s")

    @functools.partial(
        pl.kernel,
        out_type=jax.ShapeDtypeStruct((NCORES, N, 128), jnp.float32),
        mesh=mesh,
        scratch_types=[
            pltpu.VMEM((JB, B), jnp.int32),
            pltpu.VMEM((JB, B), jnp.int32),
            pltpu.VMEM((B, C), jnp.float32),
            pltpu.VMEM((B, 128), jnp.float32),
            pltpu.VMEM((B, 128), jnp.float32),
            pltpu.VMEM_SHARED((N, 128), jnp.float32),
            pltpu.SemaphoreType.DMA,
            pltpu.SemaphoreType.DMA,
            pltpu.SemaphoreType.DMA,
            pltpu.SemaphoreType.DMA,
        ],
    )
    def mp(h_hbm, e_hbm, src_hbm, dst_hbm, out_hbm,
           src_v, dst_v, ebuf, hb0, hb1, agg_sh,
           sh0, sh1, ss0, ss1):
        hbufs = (hb0, hb1)
        sems_h = (sh0, sh1)
        sems_s = (ss0, ss1)
        c = lax.axis_index("c")
        s = lax.axis_index("s")
        wid = s * NCORES + c

        zv = jnp.zeros((16,), jnp.float32)

        def zrow(r, carry):
            for k in range(8):
                hb0[r, pl.ds(k * 16, 16)] = zv
            return carry

        lax.fori_loop(0, B, zrow, 0)
        for t in range(8):
            b = s + NSUB * t

            @pl.when(b < N // B)
            def _():
                pltpu.sync_copy(hb0, agg_sh.at[pl.ds(b * B, B)])
        plsc.subcore_barrier()

        ebase = wid * EPW

        def start_fetch(p, sb, j):
            pltpu.async_copy(h_hbm.at[src_v.at[j]], hbufs[p], sems_h[p])

        def wait_fetch(p):
            pltpu.make_async_copy(
                h_hbm.at[pl.ds(0, B)], hbufs[p], sems_h[p]).wait()

        def fetch_e(sb, j):
            eoff = ebase + sb * (JB * B) + j * B
            pltpu.sync_copy(e_hbm.at[pl.ds(eoff, B)], ebuf)

        def compute(p):
            hb = hbufs[p]

            def crow(r, carry):
                for k in range(CK):
                    sl = pl.ds(16 * k, 16)
                    hb[r, sl] = jnp.maximum(hb[r, sl] + ebuf[r, sl], 0.0)
                return carry

            lax.fori_loop(0, B, crow, 0)

        def start_scatter(p, j):
            pltpu.async_copy(hbufs[p], agg_sh.at[dst_v.at[j]], sems_s[p],
                             add=True)

        def wait_scatter(p):
            pltpu.make_async_copy(
                h_hbm.at[pl.ds(0, B)], hbufs[p], sems_s[p]).wait()

        def superstep(sb, carry):
            pltpu.sync_copy(src_hbm.at[wid, sb], src_v)
            pltpu.sync_copy(dst_hbm.at[wid, sb], dst_v)
            start_fetch(0, sb, 0)
            start_fetch(1, sb, 1)

            def pair(jj, c1):
                j = 2 * jj
                fetch_e(sb, j)
                wait_fetch(0)
                compute(0)
                start_scatter(0, j)
                fetch_e(sb, j + 1)
                wait_fetch(1)
                compute(1)
                start_scatter(1, j + 1)
                wait_scatter(0)
                start_fetch(0, sb, j + 2)

                @pl.when(jj < (JB // 2) - 1)
                def _():
                    wait_scatter(1)
                    start_fetch(1, sb, j + 3)
                return c1

            lax.fori_loop(0, JB // 2, pair, 0)
            fetch_e(sb, JB - 1)
            wait_fetch(0)
            compute(0)
            pltpu.sync_copy(hbufs[0], agg_sh.at[dst_v.at[JB - 1]], add=True)
            wait_scatter(1)
            return carry

        lax.fori_loop(0, NJB, superstep, 0)
        plsc.subcore_barrier()
        for t in range(8):
            b = s + NSUB * t

            @pl.when(b < N // B)
            def _():
                sl = pl.ds(b * B, B)
                pltpu.sync_copy(agg_sh.at[sl], out_hbm.at[c].at[sl])

    return mp


_mp128 = _make_mp(128)
_mp64 = _make_mp(64)


def _dot(a, b_mat):
    return lax.dot_general(a, b_mat, (((1,), (1,)), ((), ())),
                           preferred_element_type=jnp.float32)


def _elin_body(ea_ref, w0, b0, w1, b1, w2, b2, e0_ref, e1_ref, e2_ref):
    ea = ea_ref[...]
    e0_ref[...] = _dot(ea, w0[...]) + b0[...]
    e1_ref[...] = _dot(ea, w1[...]) + b1[...]
    e2_ref[...] = _dot(ea, w2[...]) + b2[...]


def _edge_linears(edge_attr, ws, bs):
    EC = 3200
    grid = E // EC
    full = lambda shp: pl.BlockSpec(shp, lambda i: (0,) * len(shp))
    return pl.pallas_call(
        _elin_body,
        grid=(grid,),
        in_specs=[
            pl.BlockSpec((EC, DE), lambda i: (i, 0)),
            full(ws[0].shape), full(bs[0].shape),
            full(ws[1].shape), full(bs[1].shape),
            full(ws[2].shape), full(bs[2].shape),
        ],
        out_specs=[
            pl.BlockSpec((EC, 128), lambda i: (i, 0)),
            pl.BlockSpec((EC, 64), lambda i: (i, 0)),
            pl.BlockSpec((EC, 64), lambda i: (i, 0)),
        ],
        out_shape=[
            jax.ShapeDtypeStruct((E, 128), jnp.float32),
            jax.ShapeDtypeStruct((E, 64), jnp.float32),
            jax.ShapeDtypeStruct((E, 64), jnp.float32),
        ],
    )(edge_attr, ws[0], bs[0], ws[1], bs[1], ws[2], bs[2])


def _make_mlp_body(C, pad):
    def _mlp_body(h_ref, agg_ref, w1, b1, w2, b2, out_ref):
        z = h_ref[:, :C] + agg_ref[0, :, :C] + agg_ref[1, :, :C]
        z = jnp.maximum(_dot(z, w1[...]) + b1[...], 0.0)
        o = jnp.maximum(_dot(z, w2[...]) + b2[...], 0.0)
        if pad:
            out_ref[...] = jnp.concatenate(
                [o, jnp.zeros((o.shape[0], 128 - H), jnp.float32)], axis=1)
        else:
            out_ref[...] = o
    return _mlp_body


def _node_mlp(h, agg, w1, b1, w2, b2, pad, C):
    HW = h.shape[1]
    OW = 128 if pad else H
    NC_ = 2000
    grid = N // NC_
    full = lambda shp: pl.BlockSpec(shp, lambda i: (0,) * len(shp))
    return pl.pallas_call(
        _make_mlp_body(C, pad),
        grid=(grid,),
        in_specs=[
            pl.BlockSpec((NC_, HW), lambda i: (i, 0)),
            pl.BlockSpec((NCORES, NC_, 128), lambda i: (0, i, 0)),
            full(w1.shape), full(b1.shape),
            full(w2.shape), full(b2.shape),
        ],
        out_specs=pl.BlockSpec((NC_, OW), lambda i: (i, 0)),
        out_shape=jax.ShapeDtypeStruct((N, OW), jnp.float32),
    )(h, agg, w1, b1, w2, b2)


def _pool_head_body(h_ref, batch_ref, w1, b1, w2, b2, wp1, bp1, wp2, bp2,
                    wc, bc, emb_ref, proj_ref, logits_ref):
    h = h_ref[...]
    bat = batch_ref[...]
    onehot = (bat == lax.broadcasted_iota(jnp.int32, (N, G), 1)
              ).astype(jnp.float32)
    sum_pool = lax.dot_general(onehot, h, (((0,), (0,)), ((), ())),
                               preferred_element_type=jnp.float32)
    counts = jnp.sum(onehot, axis=0)
    mean_pool = sum_pool / jnp.maximum(counts, 1.0)[:, None]

    neg = jnp.float32(-jnp.inf)
    CH = 500
    mx = jnp.full((G, H), neg, jnp.float32)
    for t in range(N // CH):
        hc = h[t * CH:(t + 1) * CH]
        oc = onehot[t * CH:(t + 1) * CH]
        masked = jnp.where(oc[:, :, None] > 0.0, hc[:, None, :], neg)
        mx = jnp.maximum(mx, jnp.max(masked, axis=0))
    g = jnp.concatenate([mean_pool, mx, sum_pool], axis=1)

    emb = jnp.maximum(_dot(g, w1[...]) + b1[...], 0.0)
    emb = _dot(emb, w2[...]) + b2[...]
    emb_ref[...] = emb
    p = jnp.maximum(_dot(emb, wp1[...]) + bp1[...], 0.0)
    proj_ref[...] = _dot(p, wp2[...]) + bp2[...]
    logits_ref[...] = _dot(emb, wc[...]) + bc[...]


def _pool_head(h, batch, p):
    args = (h, batch.reshape(N, 1),
            p["emb1"]["w"], p["emb1"]["b"].reshape(1, -1),
            p["emb2"]["w"], p["emb2"]["b"].reshape(1, -1),
            p["proj1"]["w"], p["proj1"]["b"].reshape(1, -1),
            p["proj2"]["w"], p["proj2"]["b"].reshape(1, -1),
            p["cls"]["w"], p["cls"]["b"].reshape(1, -1))
    return pl.pallas_call(
        _pool_head_body,
        out_shape=(
            jax.ShapeDtypeStruct((G, 128), jnp.float32),
            jax.ShapeDtypeStruct((G, 64), jnp.float32),
            jax.ShapeDtypeStruct((G, 10), jnp.float32),
        ),
    )(*args)


def _fold_bn(w, b, bn):
    scale = bn["g"] / jnp.sqrt(bn["v"] + 1e-5)
    wf = w * scale[:, None]
    bf = scale * (b - bn["m"]) + bn["b"]
    return wf, bf.reshape(1, -1)


def kernel(x, edge_index, edge_attr, batch, params):
    src = edge_index[0].reshape(NW, NJB, JB, B)
    dst = edge_index[1].reshape(NW, NJB, JB, B)

    convs = params["convs"]
    bns = params["bns"]
    e0, e1, e2 = _edge_linears(
        edge_attr,
        [convs[i]["edge_lin"]["w"] for i in range(3)],
        [convs[i]["edge_lin"]["b"].reshape(1, -1) for i in range(3)],
    )
    es = [e0, e1, e2]
    mps = [_mp128, _mp64, _mp64]

    h = x
    for i in range(3):
        w1, b1 = _fold_bn(convs[i]["mlp_lin1"]["w"], convs[i]["mlp_lin1"]["b"],
                          convs[i]["mlp_bn"])
        w2, b2 = _fold_bn(convs[i]["mlp_lin2"]["w"], convs[i]["mlp_lin2"]["b"],
                          bns[i])
        agg = mps[i](h, es[i], src, dst)
        h = _node_mlp(h, agg, w1, b1, w2, b2, pad=(i < 2),
                      C=(128 if i == 0 else 64))

    return _pool_head(h, batch, params)


# speedup vs baseline: 4.2535x; 1.0756x over previous
import functools

import jax
import jax.numpy as jnp
from jax import lax
from jax.experimental import pallas as pl
from jax.experimental.pallas import tpu as pltpu
from jax.experimental.pallas import tpu_sc as plsc

N = 10000
E = 320000
DE = 16
H = 64
G = 16
NCORES = 2
NSUB = 16
NW = NCORES * NSUB
EPW = E // NW
B = 80
NJ = EPW // B
JB = 25
NJB = NJ // JB
RPT = N // NW
RSUB = N // NSUB


def _make_mp(C):
    CK = C // 16
    double_e = C == 64
    mesh = plsc.VectorSubcoreMesh(core_axis_name="c", subcore_axis_name="s")

    scratch = [
        pltpu.VMEM((JB, B), jnp.int32),
        pltpu.VMEM((JB, B), jnp.int32),
        pltpu.VMEM((B, C), jnp.float32),
        pltpu.VMEM((B, 128), jnp.float32),
        pltpu.VMEM((B, 128), jnp.float32),
        pltpu.VMEM_SHARED((N, 128), jnp.float32),
        pltpu.SemaphoreType.DMA,
        pltpu.SemaphoreType.DMA,
        pltpu.SemaphoreType.DMA,
        pltpu.SemaphoreType.DMA,
    ]
    if double_e:
        scratch.insert(3, pltpu.VMEM((B, C), jnp.float32))
        scratch.append(pltpu.SemaphoreType.DMA)
        scratch.append(pltpu.SemaphoreType.DMA)

    @functools.partial(
        pl.kernel,
        out_type=jax.ShapeDtypeStruct((NCORES, N, 128), jnp.float32),
        mesh=mesh,
        scratch_types=scratch,
    )
    def mp(h_hbm, e_hbm, src_hbm, dst_hbm, out_hbm,
           src_v, dst_v, *rest):
        if double_e:
            (eb0, eb1, hb0, hb1, agg_sh,
             sh0, sh1, ss0, ss1, se0, se1) = rest
            ebufs = (eb0, eb1)
            sems_e = (se0, se1)
        else:
            eb0, hb0, hb1, agg_sh, sh0, sh1, ss0, ss1 = rest
            ebufs = (eb0, eb0)
        hbufs = (hb0, hb1)
        sems_h = (sh0, sh1)
        sems_s = (ss0, ss1)
        c = lax.axis_index("c")
        s = lax.axis_index("s")
        wid = s * NCORES + c

        zv = jnp.zeros((16,), jnp.float32)

        def zrow(r, carry):
            for k in range(8):
                hb0[r, pl.ds(k * 16, 16)] = zv
            return carry

        lax.fori_loop(0, B, zrow, 0)
        for t in range(8):
            b = s + NSUB * t

            @pl.when(b < N // B)
            def _():
                pltpu.sync_copy(hb0, agg_sh.at[pl.ds(b * B, B)])
        plsc.subcore_barrier()

        ebase = wid * EPW

        def start_fetch(p, sb, j):
            pltpu.async_copy(h_hbm.at[src_v.at[j]], hbufs[p], sems_h[p])
            if double_e:
                eoff = ebase + sb * (JB * B) + j * B
                pltpu.async_copy(e_hbm.at[pl.ds(eoff, B)], ebufs[p],
                                 sems_e[p])

        def wait_fetch(p):
            pltpu.make_async_copy(
                h_hbm.at[pl.ds(0, B)], hbufs[p], sems_h[p]).wait()
            if double_e:
                pltpu.make_async_copy(
                    e_hbm.at[pl.ds(0, B)], ebufs[p], sems_e[p]).wait()

        def fetch_e(sb, j):
            if not double_e:
                eoff = ebase + sb * (JB * B) + j * B
                pltpu.sync_copy(e_hbm.at[pl.ds(eoff, B)], ebufs[0])

        def compute(p):
            hb = hbufs[p]
            eb = ebufs[p]

            def crow(r):
                for k in range(CK):
                    sl = pl.ds(16 * k, 16)
                    hb[r, sl] = jnp.maximum(hb[r, sl] + eb[r, sl], 0.0)

            plsc.parallel_loop(0, B, 1, unroll=4)(crow)

        def start_scatter(p, j):
            pltpu.async_copy(hbufs[p], agg_sh.at[dst_v.at[j]], sems_s[p],
                             add=True)

        def wait_scatter(p):
            pltpu.make_async_copy(
                h_hbm.at[pl.ds(0, B)], hbufs[p], sems_s[p]).wait()

        def superstep(sb, carry):
            pltpu.sync_copy(src_hbm.at[wid, sb], src_v)
            pltpu.sync_copy(dst_hbm.at[wid, sb], dst_v)
            start_fetch(0, sb, 0)
            start_fetch(1, sb, 1)

            def pair(jj, c1):
                j = 2 * jj
                fetch_e(sb, j)
                wait_fetch(0)
                compute(0)
                start_scatter(0, j)
                fetch_e(sb, j + 1)
                wait_fetch(1)
                compute(1)
                start_scatter(1, j + 1)
                wait_scatter(0)
                start_fetch(0, sb, j + 2)

                @pl.when(jj < (JB // 2) - 1)
                def _():
                    wait_scatter(1)
                    start_fetch(1, sb, j + 3)
                return c1

            lax.fori_loop(0, JB // 2, pair, 0)
            fetch_e(sb, JB - 1)
            wait_fetch(0)
            compute(0)
            pltpu.sync_copy(hbufs[0], agg_sh.at[dst_v.at[JB - 1]], add=True)
            wait_scatter(1)
            return carry

        lax.fori_loop(0, NJB, superstep, 0)
        plsc.subcore_barrier()
        for t in range(8):
            b = s + NSUB * t

            @pl.when(b < N // B)
            def _():
                sl = pl.ds(b * B, B)
                pltpu.sync_copy(agg_sh.at[sl], out_hbm.at[c].at[sl])

    return mp


_mp128 = _make_mp(128)
_mp64 = _make_mp(64)


def _dot(a, b_mat):
    return lax.dot_general(a, b_mat, (((1,), (1,)), ((), ())),
                           preferred_element_type=jnp.float32)


def _elin_body(ea_ref, w, b, e_ref):
    e_ref[...] = _dot(ea_ref[...], w[...]) + b[...]


def _edge_linear(edge_attr, w, b):
    EC = 3200
    C = w.shape[0]
    full = lambda shp: pl.BlockSpec(shp, lambda i: (0,) * len(shp))
    return pl.pallas_call(
        _elin_body,
        grid=(E // EC,),
        in_specs=[
            pl.BlockSpec((EC, DE), lambda i: (i, 0)),
            full(w.shape), full(b.shape),
        ],
        out_specs=pl.BlockSpec((EC, C), lambda i: (i, 0)),
        out_shape=jax.ShapeDtypeStruct((E, C), jnp.float32),
    )(edge_attr, w, b)


def _make_mlp_body(C, pad):
    def _mlp_body(h_ref, agg_ref, w1, b1, w2, b2, out_ref):
        z = h_ref[:, :C] + agg_ref[0, :, :C] + agg_ref[1, :, :C]
        z = jnp.maximum(_dot(z, w1[...]) + b1[...], 0.0)
        o = jnp.maximum(_dot(z, w2[...]) + b2[...], 0.0)
        if pad:
            out_ref[...] = jnp.concatenate(
                [o, jnp.zeros((o.shape[0], 128 - H), jnp.float32)], axis=1)
        else:
            out_ref[...] = o
    return _mlp_body


def _node_mlp(h, agg, w1, b1, w2, b2, pad, C):
    HW = h.shape[1]
    OW = 128 if pad else H
    NC_ = 2000
    grid = N // NC_
    full = lambda shp: pl.BlockSpec(shp, lambda i: (0,) * len(shp))
    return pl.pallas_call(
        _make_mlp_body(C, pad),
        grid=(grid,),
        in_specs=[
            pl.BlockSpec((NC_, HW), lambda i: (i, 0)),
            pl.BlockSpec((NCORES, NC_, 128), lambda i: (0, i, 0)),
            full(w1.shape), full(b1.shape),
            full(w2.shape), full(b2.shape),
        ],
        out_specs=pl.BlockSpec((NC_, OW), lambda i: (i, 0)),
        out_shape=jax.ShapeDtypeStruct((N, OW), jnp.float32),
    )(h, agg, w1, b1, w2, b2)


def _pool_head_body(h_ref, batch_ref, w1, b1, w2, b2, wp1, bp1, wp2, bp2,
                    wc, bc, emb_ref, proj_ref, logits_ref):
    h = h_ref[...]
    bat = batch_ref[...]
    onehot = (bat == lax.broadcasted_iota(jnp.int32, (N, G), 1)
              ).astype(jnp.float32)
    sum_pool = lax.dot_general(onehot, h, (((0,), (0,)), ((), ())),
                               preferred_element_type=jnp.float32)
    counts = jnp.sum(onehot, axis=0)
    mean_pool = sum_pool / jnp.maximum(counts, 1.0)[:, None]

    neg = jnp.float32(-jnp.inf)
    CH = 500
    mx = jnp.full((G, H), neg, jnp.float32)
    for t in range(N // CH):
        hc = h[t * CH:(t + 1) * CH]
        oc = onehot[t * CH:(t + 1) * CH]
        masked = jnp.where(oc[:, :, None] > 0.0, hc[:, None, :], neg)
        mx = jnp.maximum(mx, jnp.max(masked, axis=0))
    g = jnp.concatenate([mean_pool, mx, sum_pool], axis=1)

    emb = jnp.maximum(_dot(g, w1[...]) + b1[...], 0.0)
    emb = _dot(emb, w2[...]) + b2[...]
    emb_ref[...] = emb
    p = jnp.maximum(_dot(emb, wp1[...]) + bp1[...], 0.0)
    proj_ref[...] = _dot(p, wp2[...]) + bp2[...]
    logits_ref[...] = _dot(emb, wc[...]) + bc[...]


def _pool_head(h, batch, p):
    args = (h, batch.reshape(N, 1),
            p["emb1"]["w"], p["emb1"]["b"].reshape(1, -1),
            p["emb2"]["w"], p["emb2"]["b"].reshape(1, -1),
            p["proj1"]["w"], p["proj1"]["b"].reshape(1, -1),
            p["proj2"]["w"], p["proj2"]["b"].reshape(1, -1),
            p["cls"]["w"], p["cls"]["b"].reshape(1, -1))
    return pl.pallas_call(
        _pool_head_body,
        out_shape=(
            jax.ShapeDtypeStruct((G, 128), jnp.float32),
            jax.ShapeDtypeStruct((G, 64), jnp.float32),
            jax.ShapeDtypeStruct((G, 10), jnp.float32),
        ),
    )(*args)


def _fold_bn(w, b, bn):
    scale = bn["g"] / jnp.sqrt(bn["v"] + 1e-5)
    wf = w * scale[:, None]
    bf = scale * (b - bn["m"]) + bn["b"]
    return wf, bf.reshape(1, -1)


def kernel(x, edge_index, edge_attr, batch, params):
    src = edge_index[0].reshape(NW, NJB, JB, B)
    dst = edge_index[1].reshape(NW, NJB, JB, B)

    convs = params["convs"]
    bns = params["bns"]
    mps = [_mp128, _mp64, _mp64]

    h = x
    for i in range(3):
        ei = _edge_linear(edge_attr, convs[i]["edge_lin"]["w"],
                          convs[i]["edge_lin"]["b"].reshape(1, -1))
        w1, b1 = _fold_bn(convs[i]["mlp_lin1"]["w"], convs[i]["mlp_lin1"]["b"],
                          convs[i]["mlp_bn"])
        w2, b2 = _fold_bn(convs[i]["mlp_lin2"]["w"], convs[i]["mlp_lin2"]["b"],
                          bns[i])
        agg = mps[i](h, ei, src, dst)
        h = _node_mlp(h, agg, w1, b1, w2, b2, pad=(i < 2),
                      C=(128 if i == 0 else 64))

    return _pool_head(h, batch, params)


# speedup vs baseline: 4.4078x; 1.0363x over previous
import functools

import jax
import jax.numpy as jnp
from jax import lax
from jax.experimental import pallas as pl
from jax.experimental.pallas import tpu as pltpu
from jax.experimental.pallas import tpu_sc as plsc

N = 10000
E = 320000
DE = 16
H = 64
G = 16
NCORES = 2
NSUB = 16
NW = NCORES * NSUB
EPW = E // NW
B = 80
NJ = EPW // B
JB = 25
NJB = NJ // JB
RPT = N // NW
RSUB = N // NSUB


def _make_mp(C):
    CK = C // 16
    mesh = plsc.VectorSubcoreMesh(core_axis_name="c", subcore_axis_name="s")

    scratch = [
        pltpu.VMEM((JB, B), jnp.int32),
        pltpu.VMEM((JB, B), jnp.int32),
        pltpu.VMEM((B, C // 2), jnp.int32),
        pltpu.VMEM((B, C // 2), jnp.int32),
        pltpu.VMEM((B, 128), jnp.float32),
        pltpu.VMEM((B, 128), jnp.float32),
        pltpu.VMEM_SHARED((N, 128), jnp.float32),
        pltpu.SemaphoreType.DMA,
        pltpu.SemaphoreType.DMA,
        pltpu.SemaphoreType.DMA,
        pltpu.SemaphoreType.DMA,
        pltpu.SemaphoreType.DMA,
        pltpu.SemaphoreType.DMA,
    ]

    @functools.partial(
        pl.kernel,
        out_type=jax.ShapeDtypeStruct((NCORES, N, 128), jnp.float32),
        mesh=mesh,
        scratch_types=scratch,
        compiler_params=pltpu.CompilerParams(needs_layout_passes=False),
    )
    def mp(h_hbm, e_hbm, src_hbm, dst_hbm, out_hbm,
           src_v, dst_v, eb0, eb1, hb0, hb1, agg_sh,
           sh0, sh1, ss0, ss1, se0, se1):
        ebufs = (eb0, eb1)
        sems_e = (se0, se1)
        hbufs = (hb0, hb1)
        sems_h = (sh0, sh1)
        sems_s = (ss0, ss1)
        c = lax.axis_index("c")
        s = lax.axis_index("s")
        wid = s * NCORES + c

        zv = jnp.zeros((16,), jnp.float32)

        def zrow(r, carry):
            for k in range(8):
                hb0[r, pl.ds(k * 16, 16)] = zv
            return carry

        lax.fori_loop(0, B, zrow, 0)
        for t in range(8):
            b = s + NSUB * t

            @pl.when(b < N // B)
            def _():
                pltpu.sync_copy(hb0, agg_sh.at[pl.ds(b * B, B)])
        plsc.subcore_barrier()

        ebase = wid * EPW

        def start_fetch(p, sb, j):
            pltpu.async_copy(h_hbm.at[src_v.at[j]], hbufs[p], sems_h[p])
            eoff = ebase + sb * (JB * B) + j * B
            pltpu.async_copy(e_hbm.at[pl.ds(eoff, B)], ebufs[p], sems_e[p])

        def wait_fetch(p):
            pltpu.make_async_copy(
                h_hbm.at[pl.ds(0, B)], hbufs[p], sems_h[p]).wait()
            pltpu.make_async_copy(
                e_hbm.at[pl.ds(0, B)], ebufs[p], sems_e[p]).wait()

        def fetch_e(sb, j):
            del sb, j

        def compute(p):
            hb = hbufs[p]
            eb = ebufs[p]

            def crow(r):
                for g in range(CK // 2):
                    w16 = eb[r, pl.ds(16 * g, 16)]
                    bf = plsc.bitcast(w16, jnp.bfloat16)
                    lo, hi = plsc.unpack(
                        bf, format=plsc.PackFormat.INTERLEAVED,
                        preferred_element_type=jnp.float32)
                    sl0 = pl.ds(32 * g, 16)
                    sl1 = pl.ds(32 * g + 16, 16)
                    hb[r, sl0] = jnp.maximum(hb[r, sl0] + lo, 0.0)
                    hb[r, sl1] = jnp.maximum(hb[r, sl1] + hi, 0.0)

            plsc.parallel_loop(0, B, 1, unroll=4)(crow)

        def start_scatter(p, j):
            pltpu.async_copy(hbufs[p], agg_sh.at[dst_v.at[j]], sems_s[p],
                             add=True)

        def wait_scatter(p):
            pltpu.make_async_copy(
                h_hbm.at[pl.ds(0, B)], hbufs[p], sems_s[p]).wait()

        def superstep(sb, carry):
            pltpu.sync_copy(src_hbm.at[wid, sb], src_v)
            pltpu.sync_copy(dst_hbm.at[wid, sb], dst_v)
            start_fetch(0, sb, 0)
            start_fetch(1, sb, 1)

            def pair(jj, c1):
                j = 2 * jj
                fetch_e(sb, j)
                wait_fetch(0)
                compute(0)
                start_scatter(0, j)
                fetch_e(sb, j + 1)
                wait_fetch(1)
                compute(1)
                start_scatter(1, j + 1)
                wait_scatter(0)
                start_fetch(0, sb, j + 2)

                @pl.when(jj < (JB // 2) - 1)
                def _():
                    wait_scatter(1)
                    start_fetch(1, sb, j + 3)
                return c1

            lax.fori_loop(0, JB // 2, pair, 0)
            fetch_e(sb, JB - 1)
            wait_fetch(0)
            compute(0)
            pltpu.sync_copy(hbufs[0], agg_sh.at[dst_v.at[JB - 1]], add=True)
            wait_scatter(1)
            return carry

        lax.fori_loop(0, NJB, superstep, 0)
        plsc.subcore_barrier()
        for t in range(8):
            b = s + NSUB * t

            @pl.when(b < N // B)
            def _():
                sl = pl.ds(b * B, B)
                pltpu.sync_copy(agg_sh.at[sl], out_hbm.at[c].at[sl])

    return mp


_mp128 = _make_mp(128)
_mp64 = _make_mp(64)


def _dot(a, b_mat):
    return lax.dot_general(a, b_mat, (((1,), (1,)), ((), ())),
                           preferred_element_type=jnp.float32)


def _elin_body(ea_ref, w, b, e_ref):
    e = _dot(ea_ref[...], w[...]) + b[...]
    C = e.shape[1]
    lo = jnp.concatenate(
        [e[:, 32 * g:32 * g + 16] for g in range(C // 32)], axis=1)
    hi = jnp.concatenate(
        [e[:, 32 * g + 16:32 * g + 32] for g in range(C // 32)], axis=1)
    lo16 = lax.bitcast_convert_type(lo.astype(jnp.bfloat16), jnp.uint16)
    hi16 = lax.bitcast_convert_type(hi.astype(jnp.bfloat16), jnp.uint16)
    word = lo16.astype(jnp.uint32) | (hi16.astype(jnp.uint32) << 16)
    e_ref[...] = lax.bitcast_convert_type(word, jnp.int32)


def _edge_linear(edge_attr, w, b):
    EC = 3200
    C = w.shape[0]
    full = lambda shp: pl.BlockSpec(shp, lambda i: (0,) * len(shp))
    return pl.pallas_call(
        _elin_body,
        grid=(E // EC,),
        in_specs=[
            pl.BlockSpec((EC, DE), lambda i: (i, 0)),
            full(w.shape), full(b.shape),
        ],
        out_specs=pl.BlockSpec((EC, C // 2), lambda i: (i, 0)),
        out_shape=jax.ShapeDtypeStruct((E, C // 2), jnp.int32),
    )(edge_attr, w, b)


def _make_mlp_body(C, pad):
    def _mlp_body(h_ref, agg_ref, w1, b1, w2, b2, out_ref):
        z = h_ref[:, :C] + agg_ref[0, :, :C] + agg_ref[1, :, :C]
        z = jnp.maximum(_dot(z, w1[...]) + b1[...], 0.0)
        o = jnp.maximum(_dot(z, w2[...]) + b2[...], 0.0)
        if pad:
            out_ref[...] = jnp.concatenate(
                [o, jnp.zeros((o.shape[0], 128 - H), jnp.float32)], axis=1)
        else:
            out_ref[...] = o
    return _mlp_body


def _node_mlp(h, agg, w1, b1, w2, b2, pad, C):
    HW = h.shape[1]
    OW = 128 if pad else H
    NC_ = 2000
    grid = N // NC_
    full = lambda shp: pl.BlockSpec(shp, lambda i: (0,) * len(shp))
    return pl.pallas_call(
        _make_mlp_body(C, pad),
        grid=(grid,),
        in_specs=[
            pl.BlockSpec((NC_, HW), lambda i: (i, 0)),
            pl.BlockSpec((NCORES, NC_, 128), lambda i: (0, i, 0)),
            full(w1.shape), full(b1.shape),
            full(w2.shape), full(b2.shape),
        ],
        out_specs=pl.BlockSpec((NC_, OW), lambda i: (i, 0)),
        out_shape=jax.ShapeDtypeStruct((N, OW), jnp.float32),
    )(h, agg, w1, b1, w2, b2)


def _pool_head_body(h_ref, batch_ref, w1, b1, w2, b2, wp1, bp1, wp2, bp2,
                    wc, bc, emb_ref, proj_ref, logits_ref):
    h = h_ref[...]
    bat = batch_ref[...]
    onehot = (bat == lax.broadcasted_iota(jnp.int32, (N, G), 1)
              ).astype(jnp.float32)
    sum_pool = lax.dot_general(onehot, h, (((0,), (0,)), ((), ())),
                               preferred_element_type=jnp.float32)
    counts = jnp.sum(onehot, axis=0)
    mean_pool = sum_pool / jnp.maximum(counts, 1.0)[:, None]

    neg = jnp.float32(-jnp.inf)
    CH = 500
    mx = jnp.full((G, H), neg, jnp.float32)
    for t in range(N // CH):
        hc = h[t * CH:(t + 1) * CH]
        oc = onehot[t * CH:(t + 1) * CH]
        masked = jnp.where(oc[:, :, None] > 0.0, hc[:, None, :], neg)
        mx = jnp.maximum(mx, jnp.max(masked, axis=0))
    g = jnp.concatenate([mean_pool, mx, sum_pool], axis=1)

    emb = jnp.maximum(_dot(g, w1[...]) + b1[...], 0.0)
    emb = _dot(emb, w2[...]) + b2[...]
    emb_ref[...] = emb
    p = jnp.maximum(_dot(emb, wp1[...]) + bp1[...], 0.0)
    proj_ref[...] = _dot(p, wp2[...]) + bp2[...]
    logits_ref[...] = _dot(emb, wc[...]) + bc[...]


def _pool_head(h, batch, p):
    args = (h, batch.reshape(N, 1),
            p["emb1"]["w"], p["emb1"]["b"].reshape(1, -1),
            p["emb2"]["w"], p["emb2"]["b"].reshape(1, -1),
            p["proj1"]["w"], p["proj1"]["b"].reshape(1, -1),
            p["proj2"]["w"], p["proj2"]["b"].reshape(1, -1),
            p["cls"]["w"], p["cls"]["b"].reshape(1, -1))
    return pl.pallas_call(
        _pool_head_body,
        out_shape=(
            jax.ShapeDtypeStruct((G, 128), jnp.float32),
            jax.ShapeDtypeStruct((G, 64), jnp.float32),
            jax.ShapeDtypeStruct((G, 10), jnp.float32),
        ),
    )(*args)


def _fold_bn(w, b, bn):
    scale = bn["g"] / jnp.sqrt(bn["v"] + 1e-5)
    wf = w * scale[:, None]
    bf = scale * (b - bn["m"]) + bn["b"]
    return wf, bf.reshape(1, -1)


def kernel(x, edge_index, edge_attr, batch, params):
    src = edge_index[0].reshape(NW, NJB, JB, B)
    dst = edge_index[1].reshape(NW, NJB, JB, B)

    convs = params["convs"]
    bns = params["bns"]
    mps = [_mp128, _mp64, _mp64]

    h = x
    for i in range(3):
        ei = _edge_linear(edge_attr, convs[i]["edge_lin"]["w"],
                          convs[i]["edge_lin"]["b"].reshape(1, -1))
        w1, b1 = _fold_bn(convs[i]["mlp_lin1"]["w"], convs[i]["mlp_lin1"]["b"],
                          convs[i]["mlp_bn"])
        w2, b2 = _fold_bn(convs[i]["mlp_lin2"]["w"], convs[i]["mlp_lin2"]["b"],
                          bns[i])
        agg = mps[i](h, ei, src, dst)
        h = _node_mlp(h, agg, w1, b1, w2, b2, pad=(i < 2),
                      C=(128 if i == 0 else 64))

    return _pool_head(h, batch, params)
